# Initial kernel scaffold; baseline (speedup 1.0000x reference)
import jax
import jax.numpy as jnp
from jax.experimental import pallas as pl


def kernel(target_x, target_edge_index, target_weight, gcn_W, gcn_b, gat1_W, gat1_att_src, gat1_att_dst, gat1_b, gat2_W, gat2_att_src, gat2_att_dst, gat2_b, fc1_W, fc1_b, fc2_W, fc2_b, pro_bias):
    raise NotImplementedError("write your pallas kernel here")


# speedup vs baseline: 22.1897x; 22.1897x over previous
import functools

import jax
import jax.numpy as jnp
from jax import lax
from jax.experimental import pallas as pl
from jax.experimental.pallas import tpu as pltpu
from jax.experimental.pallas import tpu_sc as plsc

N = 50000
E = 800000
D = 128
H = 2
C = 128

NP = 51200
PT = 3200
ET = E // 16
EB = 80
SB = 400
FB = 400

_i32 = jnp.int32
_f32 = jnp.float32


def _iota16():
    return lax.iota(_i32, 16)


def _splat_i(val):
    return jnp.full((16,), val, _i32)


def _bload(ref, *ixs):
    return plsc.load_gather(ref, [_splat_i(i) for i in ixs])


def _mesh():
    return plsc.VectorSubcoreMesh(core_axis_name="c", subcore_axis_name="s")


def _newton_rsqrt(v):
    i = lax.bitcast_convert_type(v, _i32)
    y = lax.bitcast_convert_type(
        jnp.full((16,), 0x5F3759DF, _i32) - (i >> 1), _f32)
    for _ in range(4):
        y = y * (1.5 - 0.5 * v * y * y)
    return y


def _sc_deg_body(cc_hbm, w_hbm, zeros1_hbm, dis_hbm,
                 deg_sp, cc80, w400, degb, disb, sem):
    core = lax.axis_index("c")
    sub = lax.axis_index("s")
    nb = sub * PT

    pltpu.sync_copy(zeros1_hbm, deg_sp.at[pl.ds(nb, PT)])
    plsc.subcore_barrier()

    def batch(k, _):
        base = sub * ET + k * SB
        pltpu.sync_copy(w_hbm.at[pl.ds(base, SB)], w400)
        for b in range(SB // EB):
            pltpu.sync_copy(cc_hbm.at[pl.ds(base + b * EB, EB)], cc80)
            pltpu.sync_copy(w400.at[pl.ds(b * EB, EB)],
                            deg_sp.at[cc80], add=True)
        return ()

    lax.fori_loop(0, ET // SB, batch, ())
    plsc.subcore_barrier()

    pltpu.sync_copy(deg_sp.at[pl.ds(nb, PT)], degb)
    def rs(i, _):
        v = degb[pl.ds(i * 16, 16)] + 1.0
        disb[pl.ds(i * 16, 16)] = _newton_rsqrt(v)
        return ()
    lax.fori_loop(0, PT // 16, rs, (), unroll=4)

    write = jnp.logical_or(
        jnp.logical_and(core == 0, sub < 8),
        jnp.logical_and(core == 1, sub >= 8))
    @pl.when(write)
    def _():
        pltpu.sync_copy(disb, dis_hbm.at[pl.ds(nb, PT)])


def _sc_deg(cc, w, zeros1):
    k = pl.kernel(
        _sc_deg_body,
        out_type=[jax.ShapeDtypeStruct((NP,), _f32)],
        mesh=_mesh(),
        compiler_params=pltpu.CompilerParams(needs_layout_passes=False, use_tc_tiling_on_sc=False),
        scratch_types=[
            pltpu.VMEM_SHARED((NP,), _f32),
            pltpu.VMEM((EB,), _i32),
            pltpu.VMEM((SB,), _f32),
            pltpu.VMEM((PT,), _f32),
            pltpu.VMEM((PT,), _f32),
            pltpu.SemaphoreType.DMA,
        ],
    )
    return k(cc, w, zeros1)[0]


def _sc_gcn_body(rr_hbm, cc_hbm, w_hbm, dis_hbm, b_hbm,
                 h0_hbm, h1_hbm, h2_hbm, h3_hbm, zeros32_hbm,
                 x1_0, x1_1, x1_2, x1_3,
                 acc_sp, rr400, cc400, w400, cc80, disr, disc, normb,
                 rows, ub, accb, bbuf, sem, sem2):
    core = lax.axis_index("c")
    sub = lax.axis_index("s")
    nb = sub * PT
    h_list = [h0_hbm, h1_hbm, h2_hbm, h3_hbm]
    x1_list = [x1_0, x1_1, x1_2, x1_3]

    def msg_pass(j):
        h_hbm = h_list[j]
        def batch(k, _):
            base = sub * ET + k * SB
            c1 = pltpu.async_copy(rr_hbm.at[pl.ds(base, SB)], rr400, sem)
            c2 = pltpu.async_copy(cc_hbm.at[pl.ds(base, SB)], cc400, sem)
            c3 = pltpu.async_copy(w_hbm.at[pl.ds(base, SB)], w400, sem)
            c1.wait(); c2.wait(); c3.wait()
            for b in range(SB // EB):
                o = b * EB
                for g in range(EB // 16):
                    cc80[pl.ds(g * 16, 16)] = cc400[pl.ds(o + g * 16, 16)]
                ridx = rr400.at[pl.ds(o, EB)]
                g1 = pltpu.async_copy(dis_hbm.at[ridx], disr, sem)
                g2 = pltpu.async_copy(dis_hbm.at[cc80], disc, sem)
                g3 = pltpu.async_copy(h_hbm.at[ridx], rows, sem2)
                g1.wait(); g2.wait()
                for g in range(EB // 16):
                    s = pl.ds(g * 16, 16)
                    normb[s] = (disr[s] * disc[s]
                                * w400[pl.ds(o + g * 16, 16)])
                g3.wait()
                def rowscale(e, _):
                    nv = _bload(normb, e)
                    ub[e, pl.ds(0, 16)] = rows[e, pl.ds(0, 16)] * nv
                    ub[e, pl.ds(16, 16)] = rows[e, pl.ds(16, 16)] * nv
                    return ()
                lax.fori_loop(0, EB, rowscale, (), unroll=8)
                pltpu.sync_copy(ub, acc_sp.at[cc80], add=True)
            return ()
        lax.fori_loop(0, ET // SB, batch, ())

    def flush(j):
        h_hbm = h_list[j]
        pltpu.sync_copy(b_hbm.at[j], bbuf)
        b0 = bbuf[pl.ds(0, 16)]
        b1 = bbuf[pl.ds(16, 16)]
        def fbatch(b, _):
            o = nb + b * EB
            f1 = pltpu.async_copy(h_hbm.at[pl.ds(o, EB)], rows, sem)
            f2 = pltpu.async_copy(dis_hbm.at[pl.ds(o, EB)], disc, sem)
            pltpu.sync_copy(acc_sp.at[pl.ds(o, EB)], accb)
            f1.wait(); f2.wait()
            def frow(e, _):
                d2v = _bload(disc, e)
                d2v = d2v * d2v
                v0 = accb[e, pl.ds(0, 16)] + d2v * rows[e, pl.ds(0, 16)] + b0
                v1 = (accb[e, pl.ds(16, 16)]
                      + d2v * rows[e, pl.ds(16, 16)] + b1)
                ub[e, pl.ds(0, 16)] = jnp.maximum(v0, 0.0)
                ub[e, pl.ds(16, 16)] = jnp.maximum(v1, 0.0)
                return ()
            lax.fori_loop(0, EB, frow, (), unroll=8)
            pltpu.sync_copy(ub, x1_list[j].at[pl.ds(o, EB)])
            return ()
        lax.fori_loop(0, PT // EB, fbatch, ())

    for jj in range(2):
        pltpu.sync_copy(zeros32_hbm, acc_sp.at[pl.ds(nb, PT)])
        plsc.subcore_barrier()
        @pl.when(core == 0)
        def _():
            msg_pass(jj)
        @pl.when(core == 1)
        def _():
            msg_pass(jj + 2)
        plsc.subcore_barrier()
        @pl.when(core == 0)
        def _():
            flush(jj)
        @pl.when(core == 1)
        def _():
            flush(jj + 2)
        plsc.subcore_barrier()


def _sc_gcn(rr, cc, w, dis, b, h_list, zeros32):
    k = pl.kernel(
        _sc_gcn_body,
        out_type=[jax.ShapeDtypeStruct((NP, 32), _f32)] * 4,
        mesh=_mesh(),
        compiler_params=pltpu.CompilerParams(
            needs_layout_passes=False, use_tc_tiling_on_sc=False),
        scratch_types=[
            pltpu.VMEM_SHARED((NP, 32), _f32),
            pltpu.VMEM((SB,), _i32),
            pltpu.VMEM((SB,), _i32),
            pltpu.VMEM((SB,), _f32),
            pltpu.VMEM((EB,), _i32),
            pltpu.VMEM((EB,), _f32),
            pltpu.VMEM((EB,), _f32),
            pltpu.VMEM((EB,), _f32),
            pltpu.VMEM((EB, 32), _f32),
            pltpu.VMEM((EB, 32), _f32),
            pltpu.VMEM((EB, 32), _f32),
            pltpu.VMEM((32,), _f32),
            pltpu.SemaphoreType.DMA,
            pltpu.SemaphoreType.DMA,
        ],
    )
    return k(rr, cc, w, dis, b, *h_list, zeros32)


def _sc_gat_body(relu, rr_hbm, cc_hbm, asadm_hbm, b_hbm,
                 hh0_hbm, hh1_hbm, hh2_hbm, hh3_hbm,
                 zeros2_hbm, zeros32_hbm,
                 xt_0, xt_1, xt_2, xt_3, p0_hbm, p1_hbm,
                 rec0_hbm, rec1_hbm,
                 s_sp, acc_sp,
                 rr400, cc400, cc80, ar, ac, p80, p400, recr,
                 rows, alphab, ub, accb, bbuf, sem, sem2):
    core = lax.axis_index("c")
    sub = lax.axis_index("s")
    nb = sub * PT
    hh_list = [hh0_hbm, hh1_hbm, hh2_hbm, hh3_hbm]
    xt_list = [xt_0, xt_1, xt_2, xt_3]
    i16 = _iota16()

    def phase1(p_hbm):
        def batch(k, _):
            base = sub * ET + k * SB
            c1 = pltpu.async_copy(rr_hbm.at[pl.ds(base, SB)], rr400, sem)
            c2 = pltpu.async_copy(cc_hbm.at[pl.ds(base, SB)], cc400, sem)
            c1.wait(); c2.wait()
            for b in range(SB // EB):
                o = b * EB
                for g in range(EB // 16):
                    cc80[pl.ds(g * 16, 16)] = cc400[pl.ds(o + g * 16, 16)]
                g1 = pltpu.async_copy(
                    asadm_hbm.at[rr400.at[pl.ds(o, EB)]], ar, sem)
                g2 = pltpu.async_copy(asadm_hbm.at[cc80], ac, sem)
                g1.wait(); g2.wait()
                for g in range(EB // 16):
                    idx = _splat_i(g * 16) + i16
                    for hd in range(2):
                        asv = plsc.load_gather(ar, [idx, _splat_i(hd)])
                        adv = plsc.load_gather(ac, [idx, _splat_i(2 + hd)])
                        mv = plsc.load_gather(ac, [idx, _splat_i(4 + hd)])
                        t = asv + adv
                        ev = jnp.maximum(t, 0.2 * t)
                        pv = jnp.exp(ev - mv)
                        plsc.store_scatter(p80, [idx, _splat_i(hd)], pv)
                pltpu.sync_copy(p80, p_hbm.at[pl.ds(base + o, EB)])
                pltpu.sync_copy(p80, s_sp.at[cc80], add=True)
            return ()
        lax.fori_loop(0, ET // SB, batch, ())

    def phase2(rec_hbm):
        def rb(b, _):
            o = nb + b * EB
            pltpu.sync_copy(s_sp.at[pl.ds(o, EB)], p80)
            for g in range(EB // 16):
                idx = _splat_i(g * 16) + i16
                for hd in range(2):
                    sv = plsc.load_gather(p80, [idx, _splat_i(hd)])
                    rv = 0.5 / (sv + 1.0)
                    plsc.store_scatter(recr, [idx, _splat_i(hd)], rv)
            pltpu.sync_copy(recr, rec_hbm.at[pl.ds(o, EB)])
            return ()
        lax.fori_loop(0, PT // EB, rb, ())

    def msg_pass(j, p_hbm, rec_hbm):
        hh_hbm = hh_list[j]
        def batch(k, _):
            base = sub * ET + k * SB
            c1 = pltpu.async_copy(rr_hbm.at[pl.ds(base, SB)], rr400, sem)
            c2 = pltpu.async_copy(cc_hbm.at[pl.ds(base, SB)], cc400, sem)
            c3 = pltpu.async_copy(p_hbm.at[pl.ds(base, SB)], p400, sem)
            c1.wait(); c2.wait(); c3.wait()
            for b in range(SB // EB):
                o = b * EB
                for g in range(EB // 16):
                    cc80[pl.ds(g * 16, 16)] = cc400[pl.ds(o + g * 16, 16)]
                ridx = rr400.at[pl.ds(o, EB)]
                g1 = pltpu.async_copy(rec_hbm.at[cc80], recr, sem)
                g2 = pltpu.async_copy(hh_hbm.at[ridx], rows, sem2)
                g1.wait()
                for g in range(EB // 16):
                    idx = _splat_i(g * 16) + i16
                    for hd in range(2):
                        pv = plsc.load_gather(
                            p400, [idx + _splat_i(o), _splat_i(hd)])
                        rv = plsc.load_gather(recr, [idx, _splat_i(hd)])
                        plsc.store_scatter(
                            alphab, [idx, _splat_i(hd)], pv * rv)
                g2.wait()
                def rowscale(e, _):
                    a0 = _bload(alphab, e, 0)
                    a1 = _bload(alphab, e, 1)
                    u0 = (rows[e, pl.ds(0, 16)] * a0
                          + rows[e, pl.ds(32, 16)] * a1)
                    u1 = (rows[e, pl.ds(16, 16)] * a0
                          + rows[e, pl.ds(48, 16)] * a1)
                    ub[e, pl.ds(0, 16)] = u0
                    ub[e, pl.ds(16, 16)] = u1
                    return ()
                lax.fori_loop(0, EB, rowscale, (), unroll=8)
                pltpu.sync_copy(ub, acc_sp.at[cc80], add=True)
            return ()
        lax.fori_loop(0, ET // SB, batch, ())

    def flush(j, rec_hbm):
        hh_hbm = hh_list[j]
        pltpu.sync_copy(b_hbm.at[j], bbuf)
        b0 = bbuf[pl.ds(0, 16)]
        b1 = bbuf[pl.ds(16, 16)]
        def fbatch(b, _):
            o = nb + b * EB
            f1 = pltpu.async_copy(hh_hbm.at[pl.ds(o, EB)], rows, sem)
            f2 = pltpu.async_copy(rec_hbm.at[pl.ds(o, EB)], recr, sem)
            pltpu.sync_copy(acc_sp.at[pl.ds(o, EB)], accb)
            f1.wait(); f2.wait()
            def frow(e, _):
                r0 = _bload(recr, e, 0)
                r1 = _bload(recr, e, 1)
                v0 = (accb[e, pl.ds(0, 16)]
                      + r0 * rows[e, pl.ds(0, 16)]
                      + r1 * rows[e, pl.ds(32, 16)] + b0)
                v1 = (accb[e, pl.ds(16, 16)]
                      + r0 * rows[e, pl.ds(16, 16)]
                      + r1 * rows[e, pl.ds(48, 16)] + b1)
                if relu:
                    v0 = jnp.maximum(v0, 0.0)
                    v1 = jnp.maximum(v1, 0.0)
                ub[e, pl.ds(0, 16)] = v0
                ub[e, pl.ds(16, 16)] = v1
                return ()
            lax.fori_loop(0, EB, frow, (), unroll=8)
            pltpu.sync_copy(ub, xt_list[j].at[pl.ds(o, EB)])
            return ()
        lax.fori_loop(0, PT // EB, fbatch, ())

    pltpu.sync_copy(zeros2_hbm, s_sp.at[pl.ds(nb, PT)])
    plsc.subcore_barrier()
    @pl.when(core == 0)
    def _():
        phase1(p0_hbm)
    @pl.when(core == 1)
    def _():
        phase1(p1_hbm)
    plsc.subcore_barrier()
    @pl.when(core == 0)
    def _():
        phase2(rec0_hbm)
    @pl.when(core == 1)
    def _():
        phase2(rec1_hbm)
    plsc.subcore_barrier()
    for jj in range(2):
        pltpu.sync_copy(zeros32_hbm, acc_sp.at[pl.ds(nb, PT)])
        plsc.subcore_barrier()
        @pl.when(core == 0)
        def _():
            msg_pass(jj, p0_hbm, rec0_hbm)
        @pl.when(core == 1)
        def _():
            msg_pass(jj + 2, p1_hbm, rec1_hbm)
        plsc.subcore_barrier()
        @pl.when(core == 0)
        def _():
            flush(jj, rec0_hbm)
        @pl.when(core == 1)
        def _():
            flush(jj + 2, rec1_hbm)
        plsc.subcore_barrier()


def _sc_gat(rr, cc, asadm, b, hh_list, zeros2, zeros32, relu):
    k = pl.kernel(
        functools.partial(_sc_gat_body, relu),
        out_type=(
            [jax.ShapeDtypeStruct((NP, 32), _f32)] * 4
            + [jax.ShapeDtypeStruct((E, 2), _f32),
               jax.ShapeDtypeStruct((E, 2), _f32),
               jax.ShapeDtypeStruct((NP, 2), _f32),
               jax.ShapeDtypeStruct((NP, 2), _f32)]
        ),
        mesh=_mesh(),
        compiler_params=pltpu.CompilerParams(
            needs_layout_passes=False, use_tc_tiling_on_sc=False),
        scratch_types=[
            pltpu.VMEM_SHARED((NP, 2), _f32),
            pltpu.VMEM_SHARED((NP, 32), _f32),
            pltpu.VMEM((SB,), _i32),
            pltpu.VMEM((SB,), _i32),
            pltpu.VMEM((EB,), _i32),
            pltpu.VMEM((EB, 8), _f32),
            pltpu.VMEM((EB, 8), _f32),
            pltpu.VMEM((EB, 2), _f32),
            pltpu.VMEM((SB, 2), _f32),
            pltpu.VMEM((EB, 2), _f32),
            pltpu.VMEM((EB, 64), _f32),
            pltpu.VMEM((EB, 2), _f32),
            pltpu.VMEM((EB, 32), _f32),
            pltpu.VMEM((EB, 32), _f32),
            pltpu.VMEM((32,), _f32),
            pltpu.SemaphoreType.DMA,
            pltpu.SemaphoreType.DMA,
        ],
    )
    outs = k(rr, cc, asadm, b, *hh_list, zeros2, zeros32)
    return outs[:4]


_BM = 1600
_GRID = NP // _BM


def _tc_gcn_proj_body(x_ref, w_ref, o0, o1, o2, o3):
    h = jnp.dot(x_ref[...], w_ref[...], preferred_element_type=_f32)
    outs = [o0, o1, o2, o3]
    for j in range(4):
        outs[j][...] = h[:, 32 * j:32 * (j + 1)]


def _tc_gcn_proj(xpad, wpad):
    return pl.pallas_call(
        _tc_gcn_proj_body,
        grid=(_GRID,),
        in_specs=[
            pl.BlockSpec((_BM, xpad.shape[1]), lambda i: (i, 0)),
            pl.BlockSpec(wpad.shape, lambda i: (0, 0)),
        ],
        out_specs=[pl.BlockSpec((_BM, 32), lambda i: (i, 0))] * 4,
        out_shape=[jax.ShapeDtypeStruct((NP, 32), _f32)] * 4,
    )(xpad, wpad)


def _tc_gat_prep_body(x0, x1, x2, x3, w_ref, av_ref, o0, o1, o2, o3, oad):
    x = jnp.concatenate([x0[...], x1[...], x2[...], x3[...]], axis=1)
    _tc_gat_prep_body_inner(x, w_ref, av_ref, o0, o1, o2, o3, oad)


def _tc_gat_prep(x_chunks, wperm, av):
    return pl.pallas_call(
        _tc_gat_prep_body,
        grid=(_GRID,),
        in_specs=([pl.BlockSpec((_BM, 32), lambda i: (i, 0))] * 4
                  + [pl.BlockSpec(wperm.shape, lambda i: (0, 0)),
                     pl.BlockSpec(av.shape, lambda i: (0, 0))]),
        out_specs=([pl.BlockSpec((_BM, 64), lambda i: (i, 0))] * 4
                   + [pl.BlockSpec((_BM, 8), lambda i: (i, 0))]),
        out_shape=([jax.ShapeDtypeStruct((NP, 64), _f32)] * 4
                   + [jax.ShapeDtypeStruct((NP, 8), _f32)]),
    )(*x_chunks, wperm, av)


def _tc_gate_prep_body(xt0, xt1, xt2, xt3, xc0, xc1, xc2, xc3,
                       f1_ref, f2_ref, bs_ref,
                       w_ref, av_ref, ox, o0, o1, o2, o3, oad):
    xt = jnp.concatenate([xt0[...], xt1[...], xt2[...], xt3[...]], axis=1)
    x = jnp.concatenate([xc0[...], xc1[...], xc2[...], xc3[...]], axis=1)
    u = (jnp.dot(xt, f1_ref[...], preferred_element_type=_f32)
         + jnp.dot(x, f2_ref[...], preferred_element_type=_f32)
         + bs_ref[0:1, :])
    z = jax.nn.sigmoid(u)
    x2 = z * xt + (1.0 - z) * x
    ox[...] = x2
    _tc_gat_prep_body_inner(x2, w_ref, av_ref, o0, o1, o2, o3, oad)


def _tc_gat_prep_body_inner(x2, w_ref, av_ref, o0, o1, o2, o3, oad):
    hh = jnp.dot(x2, w_ref[...], preferred_element_type=_f32)
    outs = [o0, o1, o2, o3]
    for j in range(4):
        outs[j][...] = hh[:, 64 * j:64 * (j + 1)]
    sc = jnp.dot(hh, av_ref[...].T, preferred_element_type=_f32)
    as0, as1 = sc[:, 0:1], sc[:, 1:2]
    ad0, ad1 = sc[:, 2:3], sc[:, 3:4]
    t0 = as0 + ad0
    t1 = as1 + ad1
    m0 = jnp.maximum(t0, 0.2 * t0)
    m1 = jnp.maximum(t1, 0.2 * t1)
    z = jnp.zeros_like(as0)
    oad[...] = jnp.concatenate([as0, as1, ad0, ad1, m0, m1, z, z], axis=1)


def _tc_gate_prep(xt_chunks, x_chunks, f1t, f2t, bsum8, wperm, av):
    return pl.pallas_call(
        _tc_gate_prep_body,
        grid=(_GRID,),
        in_specs=([pl.BlockSpec((_BM, 32), lambda i: (i, 0))] * 8
                  + [
            pl.BlockSpec(f1t.shape, lambda i: (0, 0)),
            pl.BlockSpec(f2t.shape, lambda i: (0, 0)),
            pl.BlockSpec(bsum8.shape, lambda i: (0, 0)),
            pl.BlockSpec(wperm.shape, lambda i: (0, 0)),
            pl.BlockSpec(av.shape, lambda i: (0, 0)),
        ]),
        out_specs=([pl.BlockSpec((_BM, D), lambda i: (i, 0))]
                   + [pl.BlockSpec((_BM, 64), lambda i: (i, 0))] * 4
                   + [pl.BlockSpec((_BM, 8), lambda i: (i, 0))]),
        out_shape=([jax.ShapeDtypeStruct((NP, D), _f32)]
                   + [jax.ShapeDtypeStruct((NP, 64), _f32)] * 4
                   + [jax.ShapeDtypeStruct((NP, 8), _f32)]),
    )(*xt_chunks, *x_chunks, f1t, f2t, bsum8, wperm, av)


_BMF = 2000


def _tc_gate_final_body(xt0, xt1, xt2, xt3, x_ref, f1_ref, f2_ref,
                        bs_ref, ox):
    xt = jnp.concatenate([xt0[...], xt1[...], xt2[...], xt3[...]], axis=1)
    x = x_ref[...]
    u = (jnp.dot(xt, f1_ref[...], preferred_element_type=_f32)
         + jnp.dot(x, f2_ref[...], preferred_element_type=_f32)
         + bs_ref[0:1, :])
    z = jax.nn.sigmoid(u)
    ox[...] = z * xt + (1.0 - z) * x


def _tc_gate_final(xt_chunks, x, f1t, f2t, bsum8):
    return pl.pallas_call(
        _tc_gate_final_body,
        grid=(N // _BMF,),
        in_specs=([pl.BlockSpec((_BMF, 32), lambda i: (i, 0))] * 4
                  + [
            pl.BlockSpec((_BMF, D), lambda i: (i, 0)),
            pl.BlockSpec(f1t.shape, lambda i: (0, 0)),
            pl.BlockSpec(f2t.shape, lambda i: (0, 0)),
            pl.BlockSpec(bsum8.shape, lambda i: (0, 0)),
        ]),
        out_specs=pl.BlockSpec((_BMF, D), lambda i: (i, 0)),
        out_shape=jax.ShapeDtypeStruct((N, D), _f32),
    )(*xt_chunks, x, f1t, f2t, bsum8)


def kernel(target_x, target_edge_index, target_weight, gcn_W, gcn_b,
           gat1_W, gat1_att_src, gat1_att_dst, gat1_b,
           gat2_W, gat2_att_src, gat2_att_dst, gat2_b,
           fc1_W, fc1_b, fc2_W, fc2_b, pro_bias):
    rr = target_edge_index[0]
    cc = target_edge_index[1]

    f_in = target_x.shape[1]
    xpad = jnp.zeros((NP, 64), _f32).at[:N, :f_in].set(target_x)
    wpad = jnp.zeros((64, D), _f32).at[:f_in, :].set(gcn_W)

    perm = jnp.concatenate(
        [jnp.concatenate([jnp.arange(32 * j, 32 * (j + 1)),
                          jnp.arange(C + 32 * j, C + 32 * (j + 1))])
         for j in range(4)]).astype(_i32)

    def prep_gat(w, a_s, a_d):
        wperm = w[:, perm]
        av = jnp.stack([
            jnp.concatenate([a_s[0], jnp.zeros((C,), _f32)])[perm],
            jnp.concatenate([jnp.zeros((C,), _f32), a_s[1]])[perm],
            jnp.concatenate([a_d[0], jnp.zeros((C,), _f32)])[perm],
            jnp.concatenate([jnp.zeros((C,), _f32), a_d[1]])[perm],
        ])
        return wperm, av

    w1perm, av1 = prep_gat(gat1_W, gat1_att_src, gat1_att_dst)
    w2perm, av2 = prep_gat(gat2_W, gat2_att_src, gat2_att_dst)

    f1t = fc1_W.T
    f2t = fc2_W.T
    bsum8 = jnp.broadcast_to(fc1_b + fc2_b + pro_bias[0], (8, D))

    zeros1 = jnp.zeros((PT,), _f32)
    zeros2 = jnp.zeros((PT, 2), _f32)
    zeros32 = jnp.zeros((PT, 32), _f32)

    gcn_b4 = gcn_b.reshape(4, 32)
    gat1_b4 = gat1_b.reshape(4, 32)
    gat2_b4 = gat2_b.reshape(4, 32)

    dis = _sc_deg(cc, target_weight, zeros1)
    h_list = _tc_gcn_proj(xpad, wpad)
    x1c = _sc_gcn(rr, cc, target_weight, dis, gcn_b4, list(h_list), zeros32)
    hh1 = _tc_gat_prep(list(x1c), w1perm, av1)
    xt1c = _sc_gat(rr, cc, hh1[4], gat1_b4, list(hh1[:4]), zeros2, zeros32,
                   relu=True)
    g1 = _tc_gate_prep(list(xt1c), list(x1c), f1t, f2t, bsum8, w2perm, av2)
    x2, hh2, asadm2 = g1[0], g1[1:5], g1[5]
    xt2c = _sc_gat(rr, cc, asadm2, gat2_b4, list(hh2), zeros2, zeros32,
                   relu=False)
    out = _tc_gate_final(list(xt2c), x2, f1t, f2t, bsum8)
    return out
